# baseline (speedup 1.0000x reference)
import functools

import jax
import jax.numpy as jnp
from jax import lax
from jax.experimental import pallas as pl
from jax.experimental.pallas import tpu as pltpu
from jax.experimental.pallas import tpu_sc as plsc

NUM_CORES = 2
NUM_SUBCORES = 16
NW = NUM_CORES * NUM_SUBCORES

W = 512
K = W // 128
NH = 200
NHH = NH // 8


@jax.jit
def _sc_gather(weight, idx4):
    mesh = plsc.VectorSubcoreMesh(
        core_axis_name="c", subcore_axis_name="s",
        num_cores=NUM_CORES, num_subcores=NUM_SUBCORES,
    )

    @functools.partial(
        pl.kernel,
        out_type=jax.ShapeDtypeStruct((NH, 4, 128, 8, 128), jnp.float32),
        mesh=mesh,
        scratch_types=[
            pltpu.VMEM((2, K, 8, 128), jnp.int32),
            pltpu.VMEM((2, W, 32), jnp.float32),
            pltpu.VMEM((2, 4, K, 8, 128), jnp.float32),
            [pltpu.SemaphoreType.DMA] * 2,
            [pltpu.SemaphoreType.DMA] * 2,
            pltpu.SemaphoreType.DMA,
        ],
        compiler_params=pltpu.CompilerParams(
            use_tc_tiling_on_sc=False, needs_layout_passes=False),
    )
    def k(table_hbm, idx_hbm, out_hbm, idxg_v, rows_v, trans_v,
          gsems, osems, isem):
        wid = lax.axis_index("s") * NUM_CORES + lax.axis_index("c")
        bh0 = wid * K
        iota = lax.iota(jnp.int32, 16)

        def idx_group_copy(hh, g):
            return pltpu.make_async_copy(
                idx_hbm.at[hh, pl.ds(bh0, K)], idxg_v.at[g], isem)

        def gathers(hl, g, s):
            return [pltpu.make_async_copy(
                        table_hbm.at[idxg_v.at[g, j, hl]],
                        rows_v.at[s, pl.ds(j * 128, 128)], gsems[s])
                    for j in range(K)]

        def stores(h, s):
            return [pltpu.make_async_copy(
                        trans_v.at[s, dh],
                        out_hbm.at[h, dh, pl.ds(bh0, K)], osems[s])
                    for dh in range(4)]

        idx_group_copy(0, 0).start()
        idx_group_copy(0, 0).wait()
        for d in gathers(0, 0, 0):
            d.start()
        for d in gathers(1, 0, 1):
            d.start()

        @pl.loop(0, NHH)
        def _group(hh):
            g = lax.rem(hh, 2)

            @pl.when(hh < NHH - 1)
            def _prefetch():
                idx_group_copy(hh + 1, 1 - g).start()

            for hl in range(8):
                s = hl % 2
                h = hh * 8 + hl

                for d in gathers(hl, g, s):
                    d.wait()

                def drain():
                    for d in stores(h - 2, s):
                        d.wait()
                if hl < 2:
                    pl.when(hh > 0)(drain)
                else:
                    drain()

                @pl.loop(0, 4 * K)
                def _tp(j):
                    dh = j // K
                    kk = lax.rem(j, K)
                    for dl in range(8):
                        col = jnp.full((16,), dh * 8 + dl, jnp.int32)
                        for b16 in range(8):
                            row = iota + (kk * 128 + b16 * 16)
                            v = plsc.load_gather(rows_v.at[s], [row, col])
                            trans_v[s, dh, kk, dl, pl.ds(b16 * 16, 16)] = v

                for d in stores(h, s):
                    d.start()

                if hl == 6:
                    @pl.when(hh < NHH - 1)
                    def _w():
                        idx_group_copy(hh + 1, 1 - g).wait()
                if hl < 6:
                    for d in gathers(hl + 2, g, s):
                        d.start()
                else:
                    @pl.when(hh < NHH - 1)
                    def _refill():
                        for d in gathers(hl - 6, 1 - g, s):
                            d.start()

        for d in stores(NH - 2, 0):
            d.wait()
        for d in stores(NH - 1, 1):
            d.wait()

    return k(weight, idx4)


def kernel(indices, weight):
    idx4 = indices.astype(jnp.int32).reshape(128, 128, NHH, 8)
    idx4 = idx4.transpose(2, 0, 3, 1)
    out5 = _sc_gather(weight, idx4)
    out = out5.transpose(2, 4, 0, 1, 3).reshape(16384, NH, 32)
    return out


# speedup vs baseline: 1.5269x; 1.5269x over previous
import functools

import jax
import jax.numpy as jnp
from jax import lax
from jax.experimental import pallas as pl
from jax.experimental.pallas import tpu as pltpu
from jax.experimental.pallas import tpu_sc as plsc

NUM_CORES = 2
NUM_SUBCORES = 16
NW = NUM_CORES * NUM_SUBCORES

W = 512
K = W // 128
NH = 200
NHH = NH // 8


@jax.jit
def _sc_gather(weight, idx4):
    mesh = plsc.VectorSubcoreMesh(
        core_axis_name="c", subcore_axis_name="s",
        num_cores=NUM_CORES, num_subcores=NUM_SUBCORES,
    )

    @functools.partial(
        pl.kernel,
        out_type=jax.ShapeDtypeStruct((NH, 4, 128, 8, 128), jnp.float32),
        mesh=mesh,
        scratch_types=[
            pltpu.VMEM((2, K, 8, 128), jnp.int32),
            pltpu.VMEM((2, W, 32), jnp.float32),
            pltpu.VMEM((2, 4, K, 8, 128), jnp.float32),
            [pltpu.SemaphoreType.DMA] * 2,
            [pltpu.SemaphoreType.DMA] * 2,
            pltpu.SemaphoreType.DMA,
        ],
        compiler_params=pltpu.CompilerParams(
            use_tc_tiling_on_sc=False, needs_layout_passes=False),
    )
    def k(table_hbm, idx_hbm, out_hbm, idxg_v, rows_v, trans_v,
          gsems, osems, isem):
        wid = lax.axis_index("s") * NUM_CORES + lax.axis_index("c")
        bh0 = wid * K
        iota = lax.iota(jnp.int32, 16)

        def idx_group_copy(hh, g):
            return pltpu.make_async_copy(
                idx_hbm.at[hh, pl.ds(bh0, K)], idxg_v.at[g], isem)

        def gathers(hl, g, s):
            return [pltpu.make_async_copy(
                        table_hbm.at[idxg_v.at[g, j, hl]],
                        rows_v.at[s, pl.ds(j * 128, 128)], gsems[s])
                    for j in range(K)]

        def stores(h, s):
            return [pltpu.make_async_copy(
                        trans_v.at[s, dh],
                        out_hbm.at[h, dh, pl.ds(bh0, K)], osems[s])
                    for dh in range(4)]

        idx_group_copy(0, 0).start()
        idx_group_copy(0, 0).wait()
        for d in gathers(0, 0, 0):
            d.start()
        for d in gathers(1, 0, 1):
            d.start()

        @pl.loop(0, NHH)
        def _group(hh):
            g = lax.rem(hh, 2)

            @pl.when(hh < NHH - 1)
            def _prefetch():
                idx_group_copy(hh + 1, 1 - g).start()

            for hl in range(8):
                s = hl % 2
                h = hh * 8 + hl

                for d in gathers(hl, g, s):
                    d.wait()

                def drain():
                    for d in stores(h - 2, s):
                        d.wait()
                if hl < 2:
                    pl.when(hh > 0)(drain)
                else:
                    drain()

                @plsc.parallel_loop(0, 4 * K)
                def _tp(j):
                    dh = j // K
                    kk = lax.rem(j, K)
                    for dl in range(8):
                        col = jnp.full((16,), dh * 8 + dl, jnp.int32)
                        for b16 in range(8):
                            row = iota + (kk * 128 + b16 * 16)
                            v = plsc.load_gather(rows_v.at[s], [row, col])
                            trans_v[s, dh, kk, dl, pl.ds(b16 * 16, 16)] = v

                for d in stores(h, s):
                    d.start()

                if hl == 6:
                    @pl.when(hh < NHH - 1)
                    def _w():
                        idx_group_copy(hh + 1, 1 - g).wait()
                if hl < 6:
                    for d in gathers(hl + 2, g, s):
                        d.start()
                else:
                    @pl.when(hh < NHH - 1)
                    def _refill():
                        for d in gathers(hl - 6, 1 - g, s):
                            d.start()

        for d in stores(NH - 2, 0):
            d.wait()
        for d in stores(NH - 1, 1):
            d.wait()

    return k(weight, idx4)


def kernel(indices, weight):
    idx4 = indices.astype(jnp.int32).reshape(128, 128, NHH, 8)
    idx4 = idx4.transpose(2, 0, 3, 1)
    out5 = _sc_gather(weight, idx4)
    out = out5.transpose(2, 4, 0, 1, 3).reshape(16384, NH, 32)
    return out
